# Initial kernel scaffold; baseline (speedup 1.0000x reference)
import jax
import jax.numpy as jnp
from jax.experimental import pallas as pl


def kernel(x, edge_index, edge_weight, W1, b1, W2, b2):
    raise NotImplementedError("write your pallas kernel here")


# speedup vs baseline: 2.9274x; 2.9274x over previous
import functools

import jax
import jax.numpy as jnp
from jax import lax
from jax.experimental import pallas as pl
from jax.experimental.pallas import tpu as pltpu
from jax.experimental.pallas import tpu_sc as plsc

N = 10000
D = 128
E = 320000
NC = 2
NS = 16
NW = NC * NS
E_W = 10240
E_PAD = NW * E_W
CHUNK = 128
N_CHUNKS = E_W // CHUNK
N_PAD = 10112
ROWS_W = N_PAD // NS


def _sc_spmm(h, row3, col3, w3, zeros):
    mesh = plsc.VectorSubcoreMesh(core_axis_name="c", subcore_axis_name="s")

    @functools.partial(
        pl.kernel,
        mesh=mesh,
        out_type=jax.ShapeDtypeStruct((NC, N_PAD, D), jnp.float32),
        scratch_types=[
            pltpu.VMEM((N_CHUNKS, CHUNK), jnp.int32),
            pltpu.VMEM((N_CHUNKS, CHUNK), jnp.int32),
            pltpu.VMEM((N_CHUNKS, CHUNK), jnp.float32),
            pltpu.VMEM((CHUNK, D), jnp.float32),
            pltpu.VMEM_SHARED((N_PAD, D), jnp.float32),
            pltpu.SemaphoreType.DMA,
        ],
    )
    def k(h_hbm, row_hbm, col_hbm, w_hbm, z_hbm, out_hbm,
          row_v, col_v, w_v, rows_v, acc, sem):
        cid = lax.axis_index("c")
        sid = lax.axis_index("s")
        wid = cid * NS + sid
        rbase = sid * ROWS_W

        pltpu.sync_copy(z_hbm.at[pl.ds(rbase, ROWS_W)],
                        acc.at[pl.ds(rbase, ROWS_W)])
        pltpu.sync_copy(row_hbm.at[wid], row_v)
        pltpu.sync_copy(col_hbm.at[wid], col_v)
        pltpu.sync_copy(w_hbm.at[wid], w_v)
        plsc.subcore_barrier()

        def chunk_body(c, carry):
            pltpu.async_copy(h_hbm.at[col_v.at[c]], rows_v, sem).wait()

            def scale_body(g, carry2):
                w16 = w_v[c, pl.ds(g * 16, 16)]
                dnums = lax.GatherDimensionNumbers(
                    offset_dims=(), collapsed_slice_dims=(0,),
                    start_index_map=(0,))
                for u in range(16):
                    wv = lax.gather(
                        w16, jnp.full((16, 1), u, jnp.int32), dnums, (1,),
                        mode=lax.GatherScatterMode.PROMISE_IN_BOUNDS)
                    e = g * 16 + u
                    for j in range(D // 16):
                        rows_v[e, pl.ds(16 * j, 16)] = (
                            rows_v[e, pl.ds(16 * j, 16)] * wv)
                return carry2
            lax.fori_loop(0, CHUNK // 16, scale_body, 0)

            pltpu.sync_copy(rows_v, acc.at[row_v.at[c]], add=True)
            return carry
        lax.fori_loop(0, N_CHUNKS, chunk_body, 0)

        plsc.subcore_barrier()
        pltpu.sync_copy(acc.at[pl.ds(rbase, ROWS_W)],
                        out_hbm.at[cid, pl.ds(rbase, ROWS_W)])

    return k(h, row3, col3, w3, zeros)


def _tc_linear(x, W, b):
    BLK = 1000

    def body(x_ref, w_ref, b_ref, o_ref):
        o_ref[...] = jnp.dot(x_ref[...], w_ref[...],
                             preferred_element_type=jnp.float32) + b_ref[...]

    return pl.pallas_call(
        body,
        grid=(N // BLK,),
        in_specs=[pl.BlockSpec((BLK, D), lambda i: (i, 0)),
                  pl.BlockSpec((D, D), lambda i: (0, 0)),
                  pl.BlockSpec((1, D), lambda i: (0, 0))],
        out_specs=pl.BlockSpec((BLK, D), lambda i: (i, 0)),
        out_shape=jax.ShapeDtypeStruct((N, D), jnp.float32),
    )(x, W, b.reshape(1, D))


def _tc_combine_linear(p, W, b):
    BLK = 1000

    def body(p_ref, w_ref, b_ref, o_ref):
        hb = jnp.maximum(p_ref[0] + p_ref[1], 0.0)
        o_ref[...] = jnp.dot(hb, w_ref[...],
                             preferred_element_type=jnp.float32) + b_ref[...]

    return pl.pallas_call(
        body,
        grid=(N // BLK,),
        in_specs=[pl.BlockSpec((NC, BLK, D), lambda i: (0, i, 0)),
                  pl.BlockSpec((D, D), lambda i: (0, 0)),
                  pl.BlockSpec((1, D), lambda i: (0, 0))],
        out_specs=pl.BlockSpec((BLK, D), lambda i: (i, 0)),
        out_shape=jax.ShapeDtypeStruct((N, D), jnp.float32),
    )(p, W, b.reshape(1, D))


def _tc_combine(p):
    BLK = 1000

    def body(p_ref, o_ref):
        o_ref[...] = p_ref[0] + p_ref[1]

    return pl.pallas_call(
        body,
        grid=(N // BLK,),
        in_specs=[pl.BlockSpec((NC, BLK, D), lambda i: (0, i, 0))],
        out_specs=pl.BlockSpec((BLK, D), lambda i: (i, 0)),
        out_shape=jax.ShapeDtypeStruct((N, D), jnp.float32),
    )(p)


def kernel(x, edge_index, edge_weight, W1, b1, W2, b2):
    row = edge_index[0].astype(jnp.int32)
    col = edge_index[1].astype(jnp.int32)
    pad = E_PAD - E
    row3 = jnp.concatenate([row, jnp.zeros((pad,), jnp.int32)])
    row3 = row3.reshape(NW, N_CHUNKS, CHUNK)
    col3 = jnp.concatenate([col, jnp.zeros((pad,), jnp.int32)])
    col3 = col3.reshape(NW, N_CHUNKS, CHUNK)
    w3 = jnp.concatenate([edge_weight.astype(jnp.float32),
                          jnp.zeros((pad,), jnp.float32)])
    w3 = w3.reshape(NW, N_CHUNKS, CHUNK)
    zeros = jnp.zeros((N_PAD, D), jnp.float32)

    h = _tc_linear(x, W1, b1)
    p1 = _sc_spmm(h, row3, col3, w3, zeros)
    h2 = _tc_combine_linear(p1[:, :N], W2, b2)
    p2 = _sc_spmm(h2, row3, col3, w3, zeros)
    return _tc_combine(p2[:, :N])


# speedup vs baseline: 3.7027x; 1.2649x over previous
import functools

import jax
import jax.numpy as jnp
from jax import lax
from jax.experimental import pallas as pl
from jax.experimental.pallas import tpu as pltpu
from jax.experimental.pallas import tpu_sc as plsc

N = 10000
D = 128
E = 320000
NC = 2
NS = 16
NW = NC * NS
E_W = 10240
E_PAD = NW * E_W
CHUNK = 64
NBUF = 4
PBUF = 8
N_CHUNKS = E_W // CHUNK
N_PAD = 10112
ROWS_W = N_PAD // NS


def _sc_spmm(h, pk, w4, zeros):
    mesh = plsc.VectorSubcoreMesh(core_axis_name="c", subcore_axis_name="s")

    @functools.partial(
        pl.kernel,
        mesh=mesh,
        out_type=jax.ShapeDtypeStruct((NC, N_PAD, D), jnp.float32),
        scratch_types=[
            pltpu.VMEM((PBUF, 2, CHUNK), jnp.int32),
            pltpu.VMEM((NBUF, CHUNK + 8, D), jnp.float32),
            pltpu.VMEM_SHARED((N_PAD, D), jnp.float32),
            pltpu.SemaphoreType.DMA,
            pltpu.SemaphoreType.DMA,
            pltpu.SemaphoreType.DMA,
            pltpu.SemaphoreType.DMA,
            pltpu.SemaphoreType.DMA,
            pltpu.SemaphoreType.DMA,
            pltpu.SemaphoreType.DMA,
            pltpu.SemaphoreType.DMA,
            pltpu.SemaphoreType.DMA,
            pltpu.SemaphoreType.DMA,
            pltpu.SemaphoreType.DMA,
            pltpu.SemaphoreType.DMA,
            pltpu.SemaphoreType.DMA,
            pltpu.SemaphoreType.DMA,
            pltpu.SemaphoreType.DMA,
            pltpu.SemaphoreType.DMA,
        ],
    )
    def k(h_hbm, pk_hbm, w_hbm, z_hbm, out_hbm,
          pack_v, rows_v, acc,
          c0, c1, c2, c3, c4, c5, c6, c7,
          g0, g1, g2, g3, s0, s1, s2, s3):
        csem = [c0, c1, c2, c3, c4, c5, c6, c7]
        gsem = [g0, g1, g2, g3]
        ssem = [s0, s1, s2, s3]
        cid = lax.axis_index("c")
        sid = lax.axis_index("s")
        wid = cid * NS + sid
        rbase = sid * ROWS_W

        pltpu.sync_copy(z_hbm.at[pl.ds(rbase, ROWS_W)],
                        acc.at[pl.ds(rbase, ROWS_W)])
        plsc.subcore_barrier()

        def pack_start(c, pb):
            pltpu.async_copy(pk_hbm.at[wid, c], pack_v.at[pb], csem[pb])

        def pack_wait(c, pb):
            pltpu.make_async_copy(pk_hbm.at[wid, c], pack_v.at[pb],
                                  csem[pb]).wait()

        def gather_start(c, b, pb):
            pltpu.async_copy(h_hbm.at[pack_v.at[pb, 1]],
                             rows_v.at[b, pl.ds(0, CHUNK)], gsem[b])
            pltpu.async_copy(w_hbm.at[wid, c],
                             rows_v.at[b, pl.ds(CHUNK, 1)], gsem[b])

        def gather_wait(c, b, pb):
            pltpu.make_async_copy(h_hbm.at[pack_v.at[pb, 1]],
                                  rows_v.at[b, pl.ds(0, CHUNK)],
                                  gsem[b]).wait()
            pltpu.make_async_copy(w_hbm.at[wid, c],
                                  rows_v.at[b, pl.ds(CHUNK, 1)],
                                  gsem[b]).wait()

        def scatter_start(c, b, pb):
            pltpu.async_copy(rows_v.at[b, pl.ds(0, CHUNK)],
                             acc.at[pack_v.at[pb, 0]], ssem[b], add=True)

        def scatter_wait(c, b, pb):
            pltpu.make_async_copy(rows_v.at[b, pl.ds(0, CHUNK)],
                                  acc.at[pack_v.at[pb, 0]], ssem[b]).wait()

        def scale(b, pb):
            dnums = lax.GatherDimensionNumbers(
                offset_dims=(), collapsed_slice_dims=(0,),
                start_index_map=(0,))

            def scale_body(g, carry2):
                w16 = rows_v[b, CHUNK, pl.ds(g * 16, 16)]
                for u in range(16):
                    wv = lax.gather(
                        w16, jnp.full((16, 1), u, jnp.int32), dnums, (1,),
                        mode=lax.GatherScatterMode.PROMISE_IN_BOUNDS)
                    e = g * 16 + u
                    for j in range(D // 16):
                        rows_v[b, e, pl.ds(16 * j, 16)] = (
                            rows_v[b, e, pl.ds(16 * j, 16)] * wv)
                return carry2
            lax.fori_loop(0, CHUNK // 16, scale_body, 0)

        for c in range(4):
            pack_start(c, c)
        pack_wait(0, 0)
        pack_wait(1, 1)
        gather_start(0, 0, 0)
        gather_start(1, 1, 1)

        def group_body(grp, carry):
            for k in range(PBUF):
                c = grp * PBUF + k
                b = k % NBUF
                gather_wait(c, b, k)
                scale(b, k)
                scatter_start(c, b, k)

                @pl.when(c >= 2)
                def _():
                    scatter_wait(c - 2, (b + 2) % NBUF, (k + 6) % PBUF)

                @pl.when(c + 4 < N_CHUNKS)
                def _():
                    pack_start(c + 4, (k + 4) % PBUF)

                @pl.when(c + 2 < N_CHUNKS)
                def _():
                    pack_wait(c + 2, (k + 2) % PBUF)
                    gather_start(c + 2, (b + 2) % NBUF, (k + 2) % PBUF)
            return carry
        lax.fori_loop(0, N_CHUNKS // PBUF, group_body, 0)

        for c in (N_CHUNKS - 2, N_CHUNKS - 1):
            scatter_wait(c, c % NBUF, c % PBUF)

        plsc.subcore_barrier()
        pltpu.sync_copy(acc.at[pl.ds(rbase, ROWS_W)],
                        out_hbm.at[cid, pl.ds(rbase, ROWS_W)])

    return k(h, pk, w4, zeros)


def _tc_linear(x, W, b):
    BLK = 1000

    def body(x_ref, w_ref, b_ref, o_ref):
        o_ref[...] = jnp.dot(x_ref[...], w_ref[...],
                             preferred_element_type=jnp.float32) + b_ref[...]

    return pl.pallas_call(
        body,
        grid=(N // BLK,),
        in_specs=[pl.BlockSpec((BLK, D), lambda i: (i, 0)),
                  pl.BlockSpec((D, D), lambda i: (0, 0)),
                  pl.BlockSpec((1, D), lambda i: (0, 0))],
        out_specs=pl.BlockSpec((BLK, D), lambda i: (i, 0)),
        out_shape=jax.ShapeDtypeStruct((N, D), jnp.float32),
    )(x, W, b.reshape(1, D))


def _tc_combine_linear(p, W, b):
    BLK = 1000

    def body(p_ref, w_ref, b_ref, o_ref):
        hb = jnp.maximum(p_ref[0] + p_ref[1], 0.0)
        o_ref[...] = jnp.dot(hb, w_ref[...],
                             preferred_element_type=jnp.float32) + b_ref[...]

    return pl.pallas_call(
        body,
        grid=(N // BLK,),
        in_specs=[pl.BlockSpec((NC, BLK, D), lambda i: (0, i, 0)),
                  pl.BlockSpec((D, D), lambda i: (0, 0)),
                  pl.BlockSpec((1, D), lambda i: (0, 0))],
        out_specs=pl.BlockSpec((BLK, D), lambda i: (i, 0)),
        out_shape=jax.ShapeDtypeStruct((N, D), jnp.float32),
    )(p, W, b.reshape(1, D))


def _tc_combine(p):
    BLK = 1000

    def body(p_ref, o_ref):
        o_ref[...] = p_ref[0] + p_ref[1]

    return pl.pallas_call(
        body,
        grid=(N // BLK,),
        in_specs=[pl.BlockSpec((NC, BLK, D), lambda i: (0, i, 0))],
        out_specs=pl.BlockSpec((BLK, D), lambda i: (i, 0)),
        out_shape=jax.ShapeDtypeStruct((N, D), jnp.float32),
    )(p)


def kernel(x, edge_index, edge_weight, W1, b1, W2, b2):
    row = edge_index[0].astype(jnp.int32)
    col = edge_index[1].astype(jnp.int32)
    pad = E_PAD - E
    row_p = jnp.concatenate([row, jnp.zeros((pad,), jnp.int32)])
    col_p = jnp.concatenate([col, jnp.zeros((pad,), jnp.int32)])
    w_p = jnp.concatenate([edge_weight.astype(jnp.float32),
                           jnp.zeros((pad,), jnp.float32)])
    pk = jnp.stack([row_p.reshape(NW, N_CHUNKS, CHUNK),
                    col_p.reshape(NW, N_CHUNKS, CHUNK)], axis=2)
    w4 = jnp.pad(w_p.reshape(NW, N_CHUNKS, CHUNK), ((0, 0), (0, 0), (0, D - CHUNK)))
    w4 = w4.reshape(NW, N_CHUNKS, 1, D)
    zeros = jnp.zeros((N_PAD, D), jnp.float32)

    h = _tc_linear(x, W1, b1)
    p1 = _sc_spmm(h, pk, w4, zeros)
    h2 = _tc_combine_linear(p1[:, :N], W2, b2)
    p2 = _sc_spmm(h2, pk, w4, zeros)
    return _tc_combine(p2[:, :N])


# speedup vs baseline: 3.8851x; 1.0493x over previous
import functools

import jax
import jax.numpy as jnp
from jax import lax
from jax.experimental import pallas as pl
from jax.experimental.pallas import tpu as pltpu
from jax.experimental.pallas import tpu_sc as plsc

N = 10000
D = 128
E = 320000
NC = 2
NS = 16
NW = NC * NS
E_W = 10240
E_PAD = NW * E_W
CHUNK = 64
NBUF = 4
PBUF = 8
TOT_CHUNKS = E_PAD // CHUNK
C0 = 240
C1 = 80
N_PAD = 10112
ROWS_W = N_PAD // NS


def _sc_spmm(h, pk, w4, zeros):
    mesh = plsc.VectorSubcoreMesh(core_axis_name="c", subcore_axis_name="s")

    @functools.partial(
        pl.kernel,
        mesh=mesh,
        out_type=jax.ShapeDtypeStruct((NC, N_PAD, D), jnp.float32),
        scratch_types=[
            pltpu.VMEM((PBUF, 2, CHUNK), jnp.int32),
            pltpu.VMEM((NBUF, CHUNK + 8, D), jnp.float32),
            pltpu.VMEM_SHARED((N_PAD, D), jnp.float32),
            pltpu.SemaphoreType.DMA,
            pltpu.SemaphoreType.DMA,
            pltpu.SemaphoreType.DMA,
            pltpu.SemaphoreType.DMA,
            pltpu.SemaphoreType.DMA,
            pltpu.SemaphoreType.DMA,
            pltpu.SemaphoreType.DMA,
            pltpu.SemaphoreType.DMA,
            pltpu.SemaphoreType.DMA,
            pltpu.SemaphoreType.DMA,
            pltpu.SemaphoreType.DMA,
            pltpu.SemaphoreType.DMA,
            pltpu.SemaphoreType.DMA,
            pltpu.SemaphoreType.DMA,
            pltpu.SemaphoreType.DMA,
            pltpu.SemaphoreType.DMA,
        ],
    )
    def k(h_hbm, pk_hbm, w_hbm, z_hbm, out_hbm,
          pack_v, rows_v, acc,
          c0, c1, c2, c3, c4, c5, c6, c7,
          g0, g1, g2, g3, s0, s1, s2, s3):
        csem = [c0, c1, c2, c3, c4, c5, c6, c7]
        gsem = [g0, g1, g2, g3]
        ssem = [s0, s1, s2, s3]
        cid = lax.axis_index("c")
        sid = lax.axis_index("s")
        rbase = sid * ROWS_W
        nch = jnp.where(cid == 0, C0, C1)
        ngrp = jnp.where(cid == 0, C0 // PBUF, C1 // PBUF)
        base = jnp.where(cid == 0, sid * C0, NS * C0 + sid * C1)

        pltpu.sync_copy(z_hbm.at[pl.ds(rbase, ROWS_W)],
                        acc.at[pl.ds(rbase, ROWS_W)])
        plsc.subcore_barrier()

        def pack_start(c, pb):
            pltpu.async_copy(pk_hbm.at[base + c], pack_v.at[pb], csem[pb])

        def pack_wait(c, pb):
            pltpu.make_async_copy(pk_hbm.at[base + c], pack_v.at[pb],
                                  csem[pb]).wait()

        def gather_start(c, b, pb):
            pltpu.async_copy(h_hbm.at[pack_v.at[pb, 1]],
                             rows_v.at[b, pl.ds(0, CHUNK)], gsem[b])
            pltpu.async_copy(w_hbm.at[base + c],
                             rows_v.at[b, pl.ds(CHUNK, 1)], gsem[b])

        def gather_wait(c, b, pb):
            pltpu.make_async_copy(h_hbm.at[pack_v.at[pb, 1]],
                                  rows_v.at[b, pl.ds(0, CHUNK)],
                                  gsem[b]).wait()
            pltpu.make_async_copy(w_hbm.at[base + c],
                                  rows_v.at[b, pl.ds(CHUNK, 1)],
                                  gsem[b]).wait()

        def scatter_start(c, b, pb):
            pltpu.async_copy(rows_v.at[b, pl.ds(0, CHUNK)],
                             acc.at[pack_v.at[pb, 0]], ssem[b], add=True)

        def scatter_wait(c, b, pb):
            pltpu.make_async_copy(rows_v.at[b, pl.ds(0, CHUNK)],
                                  acc.at[pack_v.at[pb, 0]], ssem[b]).wait()

        def scale(b, pb):
            dnums = lax.GatherDimensionNumbers(
                offset_dims=(), collapsed_slice_dims=(0,),
                start_index_map=(0,))

            def scale_body(g, carry2):
                w16 = rows_v[b, CHUNK, pl.ds(g * 16, 16)]
                for u in range(16):
                    wv = lax.gather(
                        w16, jnp.full((16, 1), u, jnp.int32), dnums, (1,),
                        mode=lax.GatherScatterMode.PROMISE_IN_BOUNDS)
                    e = g * 16 + u
                    for j in range(D // 16):
                        rows_v[b, e, pl.ds(16 * j, 16)] = (
                            rows_v[b, e, pl.ds(16 * j, 16)] * wv)
                return carry2
            lax.fori_loop(0, CHUNK // 16, scale_body, 0)

        for c in range(4):
            pack_start(c, c)
        pack_wait(0, 0)
        pack_wait(1, 1)
        gather_start(0, 0, 0)
        gather_start(1, 1, 1)

        def group_body(grp, carry):
            for k in range(PBUF):
                c = grp * PBUF + k
                b = k % NBUF
                gather_wait(c, b, k)
                scale(b, k)
                scatter_start(c, b, k)

                @pl.when(c >= 2)
                def _():
                    scatter_wait(c - 2, (b + 2) % NBUF, (k + 6) % PBUF)

                @pl.when(c + 4 < nch)
                def _():
                    pack_start(c + 4, (k + 4) % PBUF)

                @pl.when(c + 2 < nch)
                def _():
                    pack_wait(c + 2, (k + 2) % PBUF)
                    gather_start(c + 2, (b + 2) % NBUF, (k + 2) % PBUF)
            return carry
        lax.fori_loop(0, ngrp, group_body, 0)

        scatter_wait(nch - 2, (PBUF - 2) % NBUF, PBUF - 2)
        scatter_wait(nch - 1, (PBUF - 1) % NBUF, PBUF - 1)

        plsc.subcore_barrier()
        pltpu.sync_copy(acc.at[pl.ds(rbase, ROWS_W)],
                        out_hbm.at[cid, pl.ds(rbase, ROWS_W)])

    return k(h, pk, w4, zeros)


def _tc_linear(x, W, b):
    BLK = 1000

    def body(x_ref, w_ref, b_ref, o_ref):
        o_ref[...] = jnp.dot(x_ref[...], w_ref[...],
                             preferred_element_type=jnp.float32) + b_ref[...]

    return pl.pallas_call(
        body,
        grid=(N // BLK,),
        in_specs=[pl.BlockSpec((BLK, D), lambda i: (i, 0)),
                  pl.BlockSpec((D, D), lambda i: (0, 0)),
                  pl.BlockSpec((1, D), lambda i: (0, 0))],
        out_specs=pl.BlockSpec((BLK, D), lambda i: (i, 0)),
        out_shape=jax.ShapeDtypeStruct((N, D), jnp.float32),
    )(x, W, b.reshape(1, D))


def _tc_combine_linear(p, W, b):
    BLK = 1000

    def body(p_ref, w_ref, b_ref, o_ref):
        hb = jnp.maximum(p_ref[0] + p_ref[1], 0.0)
        o_ref[...] = jnp.dot(hb, w_ref[...],
                             preferred_element_type=jnp.float32) + b_ref[...]

    return pl.pallas_call(
        body,
        grid=(N // BLK,),
        in_specs=[pl.BlockSpec((NC, BLK, D), lambda i: (0, i, 0)),
                  pl.BlockSpec((D, D), lambda i: (0, 0)),
                  pl.BlockSpec((1, D), lambda i: (0, 0))],
        out_specs=pl.BlockSpec((BLK, D), lambda i: (i, 0)),
        out_shape=jax.ShapeDtypeStruct((N, D), jnp.float32),
    )(p, W, b.reshape(1, D))


def _tc_combine(p):
    BLK = 1000

    def body(p_ref, o_ref):
        o_ref[...] = p_ref[0] + p_ref[1]

    return pl.pallas_call(
        body,
        grid=(N // BLK,),
        in_specs=[pl.BlockSpec((NC, BLK, D), lambda i: (0, i, 0))],
        out_specs=pl.BlockSpec((BLK, D), lambda i: (i, 0)),
        out_shape=jax.ShapeDtypeStruct((N, D), jnp.float32),
    )(p)


def kernel(x, edge_index, edge_weight, W1, b1, W2, b2):
    row = edge_index[0].astype(jnp.int32)
    col = edge_index[1].astype(jnp.int32)
    pad = E_PAD - E
    row_p = jnp.concatenate([row, jnp.zeros((pad,), jnp.int32)])
    col_p = jnp.concatenate([col, jnp.zeros((pad,), jnp.int32)])
    w_p = jnp.concatenate([edge_weight.astype(jnp.float32),
                           jnp.zeros((pad,), jnp.float32)])
    pk = jnp.stack([row_p.reshape(TOT_CHUNKS, CHUNK),
                    col_p.reshape(TOT_CHUNKS, CHUNK)], axis=1)
    w4 = jnp.pad(w_p.reshape(TOT_CHUNKS, CHUNK), ((0, 0), (0, D - CHUNK)))
    w4 = w4.reshape(TOT_CHUNKS, 1, D)
    zeros = jnp.zeros((N_PAD, D), jnp.float32)

    h = _tc_linear(x, W1, b1)
    p1 = _sc_spmm(h, pk, w4, zeros)
    h2 = _tc_combine_linear(p1[:, :N], W2, b2)
    p2 = _sc_spmm(h2, pk, w4, zeros)
    return _tc_combine(p2[:, :N])
